# baseline (speedup 1.0000x reference)
import functools

import jax
import jax.numpy as jnp
from jax import lax
from jax.experimental import pallas as pl
from jax.experimental.pallas import tpu as pltpu
from jax.experimental.pallas import tpu_sc as plsc

N = 10000
DEG = 32
D = 128
LANES = 16
NC, NS = 2, 16
NW = NC * NS
N_PAD = 10240
PW = N_PAD // NW
C = 1
EC = C * DEG
NCHUNK = PW // C
NPH = 4
CPH = NCHUNK // NPH
FCH = 40
NQ = 8


def _sc_gather_mean(features, idx_chunks):
    mesh = plsc.VectorSubcoreMesh(core_axis_name="c", subcore_axis_name="s")

    @functools.partial(
        pl.kernel,
        mesh=mesh,
        out_type=jax.ShapeDtypeStruct((N_PAD, D), jnp.float32),
        scratch_types=[
            pltpu.VMEM_SHARED((N_PAD, D), jnp.float32),
            pltpu.VMEM((CPH, EC), jnp.int32),
            pltpu.VMEM((NQ * EC, D), jnp.float32),
            pltpu.VMEM((FCH * C, D), jnp.float32),
        ] + [pltpu.SemaphoreType.DMA] * NQ,
    )
    def k(feat_hbm, idx_hbm, out_hbm, feat_sh, idx_v, rows_v, out_v, *sems):
        cid = lax.axis_index("c")
        sid = lax.axis_index("s")
        wid = sid * NC + cid
        base = wid * PW

        stripe = N_PAD // NS
        pltpu.sync_copy(feat_hbm.at[pl.ds(sid * stripe, stripe)],
                        feat_sh.at[pl.ds(sid * stripe, stripe)])
        plsc.subcore_barrier()

        def quarter(ci):
            return lax.rem(ci, NQ)

        def dst(ci):
            off = pl.multiple_of(quarter(ci) * EC, EC)
            return rows_v.at[pl.ds(off, EC)]

        def start(ci):
            d = dst(ci)
            for q in range(NQ):
                @pl.when(quarter(ci) == q)
                def _(sem=sems[q]):
                    pltpu.async_copy(feat_sh.at[idx_v.at[ci]], d, sem)

        def wait(ci):
            d = dst(ci)
            for q in range(NQ):
                @pl.when(quarter(ci) == q)
                def _(sem=sems[q]):
                    pltpu.make_async_copy(feat_sh.at[idx_v.at[ci]], d,
                                          sem).wait()

        for ph in range(NPH):
            pltpu.sync_copy(
                idx_hbm.at[pl.ds(wid * NCHUNK + ph * CPH, CPH)], idx_v)
            for j in range(NQ - 1):
                start(j)

            def chunk_body(ci, carry, ph=ph):
                @pl.when(ci + (NQ - 1) < CPH)
                def _():
                    start(ci + (NQ - 1))

                wait(ci)

                rb = quarter(ci) * EC
                loc = lax.rem(ci, FCH)
                for c in range(C):
                    row_out = loc * C + c
                    for g in range(D // LANES):
                        sl = pl.ds(g * LANES, LANES)
                        acc = rows_v[rb + c * DEG, sl]
                        for r in range(1, DEG):
                            acc = acc + rows_v[rb + c * DEG + r, sl]
                        out_v[row_out, sl] = acc * (1.0 / DEG)

                @pl.when(loc == FCH - 1)
                def _():
                    node0 = pl.multiple_of(
                        base + ph * CPH * C + (ci - (FCH - 1)) * C, FCH * C)
                    pltpu.sync_copy(out_v, out_hbm.at[pl.ds(node0, FCH * C)])

                return carry

            lax.fori_loop(0, CPH, chunk_body, 0)

    return k(features, idx_chunks)


def _tc_dense(features, agg, w_top, w_bot):
    bn = 2000

    def body(f_ref, a_ref, wt_ref, wb_ref, o_ref):
        acc = jnp.dot(f_ref[...], wt_ref[...], preferred_element_type=jnp.float32)
        acc = acc + jnp.dot(a_ref[...], wb_ref[...], preferred_element_type=jnp.float32)
        o_ref[...] = jnp.maximum(acc, 0.0)

    return pl.pallas_call(
        body,
        grid=(N // bn,),
        in_specs=[
            pl.BlockSpec((bn, D), lambda i: (i, 0)),
            pl.BlockSpec((bn, D), lambda i: (i, 0)),
            pl.BlockSpec((D, D), lambda i: (0, 0)),
            pl.BlockSpec((D, D), lambda i: (0, 0)),
        ],
        out_specs=pl.BlockSpec((bn, D), lambda i: (i, 0)),
        out_shape=jax.ShapeDtypeStruct((N, D), jnp.float32),
    )(features, agg, w_top, w_bot)


def kernel(features, edge_look_up, kernel):
    idx = edge_look_up.astype(jnp.int32).reshape(-1)
    idx = jnp.pad(idx, (0, N_PAD * DEG - idx.shape[0]))
    feat_pad = jnp.pad(features, ((0, N_PAD - N), (0, 0)))
    agg = _sc_gather_mean(feat_pad, idx.reshape(N_PAD * DEG // EC, EC))[:N]
    return _tc_dense(features, agg, kernel[:D], kernel[D:])


# speedup vs baseline: 1.0858x; 1.0858x over previous
import functools

import jax
import jax.numpy as jnp
from jax import lax
from jax.experimental import pallas as pl
from jax.experimental.pallas import tpu as pltpu
from jax.experimental.pallas import tpu_sc as plsc

N = 10000
DEG = 32
D = 128
LANES = 16
NC, NS = 2, 16
NW = NC * NS
N_PAD = 10240
PW = N_PAD // NW
C = 2
EC = C * DEG
NCHUNK = PW // C
NPH = 2
CPH = NCHUNK // NPH
FCH = 20
NQ = 4


def _sc_gather_mean(features, idx_chunks):
    mesh = plsc.VectorSubcoreMesh(core_axis_name="c", subcore_axis_name="s")

    @functools.partial(
        pl.kernel,
        mesh=mesh,
        out_type=jax.ShapeDtypeStruct((N_PAD, D), jnp.float32),
        scratch_types=[
            pltpu.VMEM_SHARED((N_PAD, D), jnp.float32),
            pltpu.VMEM((CPH, EC), jnp.int32),
            pltpu.VMEM((NQ * EC, D), jnp.float32),
            pltpu.VMEM((FCH * C, D), jnp.float32),
            pltpu.SemaphoreType.DMA,
            pltpu.SemaphoreType.DMA,
            pltpu.SemaphoreType.DMA,
            pltpu.SemaphoreType.DMA,
        ],
    )
    def k(feat_hbm, idx_hbm, out_hbm, feat_sh, idx_v, rows_v, out_v,
          sem0, sem1, sem2, sem3):
        cid = lax.axis_index("c")
        sid = lax.axis_index("s")
        wid = sid * NC + cid
        base = wid * PW

        stripe = N_PAD // NS
        pltpu.sync_copy(feat_hbm.at[pl.ds(sid * stripe, stripe)],
                        feat_sh.at[pl.ds(sid * stripe, stripe)])
        plsc.subcore_barrier()

        def quarter(ci):
            return lax.rem(ci, NQ)

        def dst(ci):
            off = pl.multiple_of(quarter(ci) * EC, EC)
            return rows_v.at[pl.ds(off, EC)]

        def start(ci):
            d = dst(ci)
            for q, sem in ((0, sem0), (1, sem1), (2, sem2), (3, sem3)):
                @pl.when(quarter(ci) == q)
                def _(sem=sem):
                    pltpu.async_copy(feat_sh.at[idx_v.at[ci]], d, sem)

        def wait(ci):
            d = dst(ci)
            for q, sem in ((0, sem0), (1, sem1), (2, sem2), (3, sem3)):
                @pl.when(quarter(ci) == q)
                def _(sem=sem):
                    pltpu.make_async_copy(feat_sh.at[idx_v.at[ci]], d,
                                          sem).wait()

        for ph in range(NPH):
            pltpu.sync_copy(
                idx_hbm.at[pl.ds(wid * NCHUNK + ph * CPH, CPH)], idx_v)
            for j in range(NQ - 1):
                start(j)

            def chunk_body(ci, carry, ph=ph):
                @pl.when(ci + (NQ - 1) < CPH)
                def _():
                    start(ci + (NQ - 1))

                wait(ci)

                rb = quarter(ci) * EC
                loc = lax.rem(ci, FCH)
                for c in range(C):
                    row_out = loc * C + c
                    for g in range(D // LANES):
                        sl = pl.ds(g * LANES, LANES)
                        acc = rows_v[rb + c * DEG, sl]
                        for r in range(1, DEG):
                            acc = acc + rows_v[rb + c * DEG + r, sl]
                        out_v[row_out, sl] = acc * (1.0 / DEG)

                @pl.when(loc == FCH - 1)
                def _():
                    node0 = pl.multiple_of(
                        base + ph * CPH * C + (ci - (FCH - 1)) * C, FCH * C)
                    pltpu.sync_copy(out_v, out_hbm.at[pl.ds(node0, FCH * C)])

                return carry

            lax.fori_loop(0, CPH, chunk_body, 0)

    return k(features, idx_chunks)


def _tc_dense(features, agg, w_top, w_bot):
    bn = 2000

    def body(f_ref, a_ref, wt_ref, wb_ref, o_ref):
        acc = jnp.dot(f_ref[...], wt_ref[...], preferred_element_type=jnp.float32)
        acc = acc + jnp.dot(a_ref[...], wb_ref[...], preferred_element_type=jnp.float32)
        o_ref[...] = jnp.maximum(acc, 0.0)

    return pl.pallas_call(
        body,
        grid=(N // bn,),
        in_specs=[
            pl.BlockSpec((bn, D), lambda i: (i, 0)),
            pl.BlockSpec((bn, D), lambda i: (i, 0)),
            pl.BlockSpec((D, D), lambda i: (0, 0)),
            pl.BlockSpec((D, D), lambda i: (0, 0)),
        ],
        out_specs=pl.BlockSpec((bn, D), lambda i: (i, 0)),
        out_shape=jax.ShapeDtypeStruct((N, D), jnp.float32),
    )(features, agg, w_top, w_bot)


def kernel(features, edge_look_up, kernel):
    idx = edge_look_up.astype(jnp.int32).reshape(-1)
    idx = jnp.pad(idx, (0, N_PAD * DEG - idx.shape[0]))
    feat_pad = jnp.pad(features, ((0, N_PAD - N), (0, 0)))
    agg = _sc_gather_mean(feat_pad, idx.reshape(N_PAD * DEG // EC, EC))[:N]
    return _tc_dense(features, agg, kernel[:D], kernel[D:])


# speedup vs baseline: 1.4225x; 1.3101x over previous
import functools

import jax
import jax.numpy as jnp
from jax import lax
from jax.experimental import pallas as pl
from jax.experimental.pallas import tpu as pltpu
from jax.experimental.pallas import tpu_sc as plsc

N = 10000
DEG = 32
D = 128
LANES = 16
NC, NS = 2, 16
NW = NC * NS
N_PAD = 10240
PW = N_PAD // NW
C = 2
EC = C * DEG
NCHUNK = PW // C
NPH = 2
CPH = NCHUNK // NPH
FCH = 20
NQ = 4


def _sc_gather_mean(features, idx_chunks):
    mesh = plsc.VectorSubcoreMesh(core_axis_name="c", subcore_axis_name="s")

    @functools.partial(
        pl.kernel,
        mesh=mesh,
        out_type=jax.ShapeDtypeStruct((N_PAD, D), jnp.float32),
        scratch_types=[
            pltpu.VMEM_SHARED((N_PAD, D), jnp.float32),
            pltpu.VMEM((CPH, EC), jnp.int32),
            pltpu.VMEM((NQ * EC, D), jnp.float32),
            pltpu.VMEM((FCH * C, D), jnp.float32),
            pltpu.SemaphoreType.DMA,
            pltpu.SemaphoreType.DMA,
            pltpu.SemaphoreType.DMA,
            pltpu.SemaphoreType.DMA,
        ],
    )
    def k(feat_hbm, idx_hbm, out_hbm, feat_sh, idx_v, rows_v, out_v,
          sem0, sem1, sem2, sem3):
        cid = lax.axis_index("c")
        sid = lax.axis_index("s")
        wid = sid * NC + cid
        base = wid * PW

        stripe = N_PAD // NS

        @pl.when(sid < NS - 1)
        def _():
            pltpu.sync_copy(feat_hbm.at[pl.ds(sid * stripe, stripe)],
                            feat_sh.at[pl.ds(sid * stripe, stripe)])

        @pl.when(sid == NS - 1)
        def _():
            last = N - (NS - 1) * stripe
            pltpu.sync_copy(feat_hbm.at[pl.ds((NS - 1) * stripe, last)],
                            feat_sh.at[pl.ds((NS - 1) * stripe, last)])

        plsc.subcore_barrier()

        def quarter(ci):
            return lax.rem(ci, NQ)

        def dst(ci):
            off = pl.multiple_of(quarter(ci) * EC, EC)
            return rows_v.at[pl.ds(off, EC)]

        def start(ci):
            d = dst(ci)
            for q, sem in ((0, sem0), (1, sem1), (2, sem2), (3, sem3)):
                @pl.when(quarter(ci) == q)
                def _(sem=sem):
                    pltpu.async_copy(feat_sh.at[idx_v.at[ci]], d, sem)

        def wait(ci):
            d = dst(ci)
            for q, sem in ((0, sem0), (1, sem1), (2, sem2), (3, sem3)):
                @pl.when(quarter(ci) == q)
                def _(sem=sem):
                    pltpu.make_async_copy(feat_sh.at[idx_v.at[ci]], d,
                                          sem).wait()

        for ph in range(NPH):
            pltpu.sync_copy(
                idx_hbm.at[pl.ds(wid * NCHUNK + ph * CPH, CPH)], idx_v)
            for j in range(NQ - 1):
                start(j)

            def chunk_body(ci, carry, ph=ph):
                @pl.when(ci + (NQ - 1) < CPH)
                def _():
                    start(ci + (NQ - 1))

                wait(ci)

                rb = quarter(ci) * EC
                loc = lax.rem(ci, FCH)
                for c in range(C):
                    row_out = loc * C + c
                    for g in range(D // LANES):
                        sl = pl.ds(g * LANES, LANES)
                        vals = [rows_v[rb + c * DEG + r, sl]
                                for r in range(DEG)]
                        while len(vals) > 1:
                            vals = [a + b for a, b in
                                    zip(vals[::2], vals[1::2])]
                        out_v[row_out, sl] = vals[0] * (1.0 / DEG)

                @pl.when(loc == FCH - 1)
                def _():
                    node0 = pl.multiple_of(
                        base + ph * CPH * C + (ci - (FCH - 1)) * C, FCH * C)
                    pltpu.sync_copy(out_v, out_hbm.at[pl.ds(node0, FCH * C)])

                return carry

            lax.fori_loop(0, CPH, chunk_body, 0)

    return k(features, idx_chunks)


def _tc_dense(features, agg, w_top, w_bot):
    bn = 2000

    def body(f_ref, a_ref, wt_ref, wb_ref, o_ref):
        acc = jnp.dot(f_ref[...], wt_ref[...], preferred_element_type=jnp.float32)
        acc = acc + jnp.dot(a_ref[...], wb_ref[...], preferred_element_type=jnp.float32)
        o_ref[...] = jnp.maximum(acc, 0.0)

    return pl.pallas_call(
        body,
        grid=(N // bn,),
        in_specs=[
            pl.BlockSpec((bn, D), lambda i: (i, 0)),
            pl.BlockSpec((bn, D), lambda i: (i, 0)),
            pl.BlockSpec((D, D), lambda i: (0, 0)),
            pl.BlockSpec((D, D), lambda i: (0, 0)),
        ],
        out_specs=pl.BlockSpec((bn, D), lambda i: (i, 0)),
        out_shape=jax.ShapeDtypeStruct((N, D), jnp.float32),
    )(features, agg, w_top, w_bot)


def kernel(features, edge_look_up, kernel):
    idx = edge_look_up.astype(jnp.int32).reshape(-1)
    idx = jnp.pad(idx, (0, N_PAD * DEG - idx.shape[0]))
    agg = _sc_gather_mean(features, idx.reshape(N_PAD * DEG // EC, EC))
    return _tc_dense(features, agg, kernel[:D], kernel[D:])
